# baseline (speedup 1.0000x reference)
import functools

import jax
import jax.numpy as jnp
from jax import lax
from jax.experimental import pallas as pl
from jax.experimental.pallas import tpu as pltpu
from jax.experimental.pallas import tpu_sc as plsc

B, T, D = 2, 2048, 768
K = 16
NUM_CLASSES = 1000
G = 63
TOP_M = 8

NCHUNK = T // 16
DCHUNK = D // 16


def _scores_body(e_ref, q_ref, out_ref):
    E2 = e_ref[0]
    sq = jnp.where(q_ref[...] >= 0, 1.0, -1.0).astype(jnp.float32)
    out_ref[...] = lax.dot_general(
        sq, E2, (((1,), (1,)), ((), ())),
        preferred_element_type=jnp.float32,
        precision=lax.Precision.HIGHEST,
    )


def _sc_body(scores_hbm, e_hbm, c_hbm, cb_hbm, out_hbm,
             s_v, idx_v, rows_v, g_v, cw_v, cb_v, out_v, sem_c, sem_g):
    cid = lax.axis_index("c")
    sid = lax.axis_index("s")
    lanes = lax.broadcasted_iota(jnp.int32, (16,), 0)
    NEG = jnp.float32(-jnp.inf)
    BIG = jnp.int32(1 << 30)
    bb = cid
    w = bb * K + sid

    ccopy = pltpu.async_copy(c_hbm.at[sid], cw_v.at[pl.ds(0, G * D)], sem_c)
    pltpu.sync_copy(cb_hbm.at[sid], cb_v)
    pltpu.sync_copy(scores_hbm.at[w], s_v)

    NACC = 8

    def combine(xv, xi, yv, yi):
        idx = jnp.where(
            yv > xv, yi, jnp.where(xv > yv, xi, jnp.minimum(xi, yi))
        )
        return jnp.maximum(xv, yv), idx

    idx_acc = jnp.full((16,), bb * T, jnp.int32)
    for m in range(TOP_M):
        def fold(jj, carry):
            out = list(carry)
            for a in range(NACC):
                mval, midx = out[2 * a], out[2 * a + 1]
                off = jj * (16 * NACC) + a * 16
                v = s_v[pl.ds(off, 16)]
                upd = v > mval
                out[2 * a] = jnp.where(upd, v, mval)
                out[2 * a + 1] = jnp.where(upd, off + lanes, midx)
            return tuple(out)

        init = []
        for a in range(NACC):
            init += [jnp.full((16,), NEG, jnp.float32),
                     jnp.full((16,), BIG, jnp.int32)]
        res = lax.fori_loop(0, NCHUNK // NACC, fold, tuple(init))
        vs = [res[2 * a] for a in range(NACC)]
        is_ = [res[2 * a + 1] for a in range(NACC)]
        while len(vs) > 1:
            nv, ni = [], []
            for a in range(0, len(vs), 2):
                cv, ci = combine(vs[a], is_[a], vs[a + 1], is_[a + 1])
                nv.append(cv)
                ni.append(ci)
            vs, is_ = nv, ni
        mval, midx = vs[0], is_[0]
        mx = mval[0]
        tidx = midx[0]
        for l in range(1, 16):
            v = mval[l]
            i = midx[l]
            take = (v > mx) | ((v == mx) & (i < tidx))
            mx = jnp.where(take, v, mx)
            tidx = jnp.where(take, i, tidx)
        jc = tidx // 16
        lc = tidx - jc * 16
        chunk = s_v[pl.ds(jc * 16, 16)]
        s_v[pl.ds(jc * 16, 16)] = jnp.where(lanes == lc, NEG, chunk)
        idx_acc = jnp.where(lanes == m, bb * T + tidx, idx_acc)

    idx_v[...] = idx_acc
    pltpu.async_copy(e_hbm.at[idx_v.at[pl.ds(0, TOP_M)]], rows_v, sem_g).wait()

    for j in range(DCHUNK):
        acc = rows_v[0, pl.ds(j * 16, 16)]
        for r in range(1, TOP_M):
            acc = acc + rows_v[r, pl.ds(j * 16, 16)]
        g_v[pl.ds(j * 16, 16)] = acc

    ccopy.wait()

    def quad(qb, carry):
        outs = list(carry)
        accs = [jnp.zeros((16,), jnp.float32) for _ in range(4)]
        base = qb * 4
        for j in range(DCHUNK):
            gv = g_v[pl.ds(j * 16, 16)]
            ngv = -gv
            for s in range(4):
                cw = cw_v[pl.ds((base + s) * D + j * 16, 16)]
                accs[s] = accs[s] + jnp.where(cw >= 0, gv, ngv)
        for s in range(4):
            gg = base + s
            h = accs[s] + lax.rev(accs[s], (0,))
            tot = h[0]
            for l in range(1, 8):
                tot = tot + h[l]
            for q in range(4):
                outs[q] = jnp.where(lanes == gg - q * 16, tot, outs[q])
        return tuple(outs)

    zero = jnp.zeros((16,), jnp.float32)
    res = lax.fori_loop(0, 16, quad, (zero, zero, zero, zero))
    for q in range(4):
        out_v[pl.ds(q * 16, 16)] = res[q] + cb_v[pl.ds(q * 16, 16)]

    pltpu.sync_copy(out_v, out_hbm.at[w])


def kernel(E, q_weight, c_weight, q_bias, cls_bias):
    del q_bias
    scores = pl.pallas_call(
        _scores_body,
        grid=(B,),
        in_specs=[
            pl.BlockSpec((1, T, D), lambda b: (b, 0, 0)),
            pl.BlockSpec((K, D), lambda b: (0, 0)),
        ],
        out_specs=pl.BlockSpec((K, T), lambda b: (b, 0)),
        out_shape=jax.ShapeDtypeStruct((B * K, T), jnp.float32),
    )(E, q_weight)

    e_flat = E.reshape(B * T, D)
    c_flat = c_weight.reshape(K, G * D)
    cb_pad = jnp.zeros((K, 64), jnp.float32).at[:, :G].set(cls_bias)

    mesh = plsc.VectorSubcoreMesh(core_axis_name="c", subcore_axis_name="s")
    sck = functools.partial(
        pl.kernel,
        mesh=mesh,
        out_type=jax.ShapeDtypeStruct((B * K, 64), jnp.float32),
        scratch_types=[
            pltpu.VMEM((T,), jnp.float32),
            pltpu.VMEM((16,), jnp.int32),
            pltpu.VMEM((TOP_M, D), jnp.float32),
            pltpu.VMEM((D,), jnp.float32),
            pltpu.VMEM((G * D + D,), jnp.float32),
            pltpu.VMEM((64,), jnp.float32),
            pltpu.VMEM((64,), jnp.float32),
            pltpu.SemaphoreType.DMA,
            pltpu.SemaphoreType.DMA,
        ],
    )(_sc_body)

    lg = sck(scores, e_flat, c_flat, cb_pad)
    return lg.reshape(B, K, 64)[:, :, :G].reshape(B, K * G)[:, :NUM_CLASSES]


# speedup vs baseline: 1.3447x; 1.3447x over previous
import functools

import jax
import jax.numpy as jnp
from jax import lax
from jax.experimental import pallas as pl
from jax.experimental.pallas import tpu as pltpu
from jax.experimental.pallas import tpu_sc as plsc

B, T, D = 2, 2048, 768
K = 16
NUM_CLASSES = 1000
G = 63
TOP_M = 8

NCHUNK = T // 16
DCHUNK = D // 16


def _scores_body(e_ref, q_ref, out_ref):
    E2 = e_ref[0]
    sq = jnp.where(q_ref[...] >= 0, 1.0, -1.0).astype(jnp.float32)
    out_ref[...] = lax.dot_general(
        sq, E2, (((1,), (1,)), ((), ())),
        preferred_element_type=jnp.float32,
        precision=lax.Precision.HIGHEST,
    )


def _sc_body(scores_hbm, e_hbm, g_hbm, s_v, idx_v, rows_v, g_v, sem_g):
    cid = lax.axis_index("c")
    sid = lax.axis_index("s")
    lanes = lax.broadcasted_iota(jnp.int32, (16,), 0)
    NEG = jnp.float32(-jnp.inf)
    BIG = jnp.int32(1 << 30)
    _sc_one(scores_hbm, e_hbm, g_hbm, s_v, idx_v, rows_v, g_v, sem_g,
            cid, sid, lanes, NEG, BIG)


def _sc_one(scores_hbm, e_hbm, g_hbm, s_v, idx_v, rows_v, g_v, sem_g,
            bb, sid, lanes, NEG, BIG):
    w = bb * K + sid

    pltpu.sync_copy(scores_hbm.at[w], s_v)

    NACC = 8

    def combine(xv, xi, yv, yi):
        idx = jnp.where(
            yv > xv, yi, jnp.where(xv > yv, xi, jnp.minimum(xi, yi))
        )
        return jnp.maximum(xv, yv), idx

    idx_acc = jnp.full((16,), bb * T, jnp.int32)
    for m in range(TOP_M):
        def fold(jj, carry):
            out = list(carry)
            for a in range(NACC):
                mval, midx = out[2 * a], out[2 * a + 1]
                off = jj * (16 * NACC) + a * 16
                v = s_v[pl.ds(off, 16)]
                upd = v > mval
                out[2 * a] = jnp.where(upd, v, mval)
                out[2 * a + 1] = jnp.where(upd, off + lanes, midx)
            return tuple(out)

        init = []
        for a in range(NACC):
            init += [jnp.full((16,), NEG, jnp.float32),
                     jnp.full((16,), BIG, jnp.int32)]
        res = lax.fori_loop(0, NCHUNK // NACC, fold, tuple(init))
        vs = [res[2 * a] for a in range(NACC)]
        is_ = [res[2 * a + 1] for a in range(NACC)]
        while len(vs) > 1:
            nv, ni = [], []
            for a in range(0, len(vs), 2):
                cv, ci = combine(vs[a], is_[a], vs[a + 1], is_[a + 1])
                nv.append(cv)
                ni.append(ci)
            vs, is_ = nv, ni
        mval, midx = vs[0], is_[0]
        mx = mval[0]
        tidx = midx[0]
        for l in range(1, 16):
            v = mval[l]
            i = midx[l]
            take = (v > mx) | ((v == mx) & (i < tidx))
            mx = jnp.where(take, v, mx)
            tidx = jnp.where(take, i, tidx)
        jc = tidx // 16
        lc = tidx - jc * 16
        chunk = s_v[pl.ds(jc * 16, 16)]
        s_v[pl.ds(jc * 16, 16)] = jnp.where(lanes == lc, NEG, chunk)
        idx_acc = jnp.where(lanes == m, bb * T + tidx, idx_acc)

    idx_v[...] = idx_acc
    pltpu.async_copy(e_hbm.at[idx_v.at[pl.ds(0, TOP_M)]], rows_v, sem_g).wait()

    for j in range(DCHUNK):
        acc = rows_v[0, pl.ds(j * 16, 16)]
        for r in range(1, TOP_M):
            acc = acc + rows_v[r, pl.ds(j * 16, 16)]
        g_v[pl.ds(j * 16, 16)] = acc

    pltpu.sync_copy(g_v, g_hbm.at[w])


def _contract_body(g_ref, c_ref, cb_ref, out_ref):
    g = g_ref[...]
    signed = jnp.where(c_ref[...] >= 0, g[:, None, :], -g[:, None, :])
    out_ref[0] = jnp.sum(signed, axis=-1) + cb_ref[...]


def kernel(E, q_weight, c_weight, q_bias, cls_bias):
    del q_bias
    scores = pl.pallas_call(
        _scores_body,
        grid=(B,),
        in_specs=[
            pl.BlockSpec((1, T, D), lambda b: (b, 0, 0)),
            pl.BlockSpec((K, D), lambda b: (0, 0)),
        ],
        out_specs=pl.BlockSpec((K, T), lambda b: (b, 0)),
        out_shape=jax.ShapeDtypeStruct((B * K, T), jnp.float32),
    )(E, q_weight)

    e_flat = E.reshape(B * T, D)

    mesh = plsc.VectorSubcoreMesh(core_axis_name="c", subcore_axis_name="s")
    sck = functools.partial(
        pl.kernel,
        mesh=mesh,
        out_type=jax.ShapeDtypeStruct((B * K, D), jnp.float32),
        scratch_types=[
            pltpu.VMEM((T,), jnp.float32),
            pltpu.VMEM((16,), jnp.int32),
            pltpu.VMEM((TOP_M, D), jnp.float32),
            pltpu.VMEM((D,), jnp.float32),
            pltpu.SemaphoreType.DMA,
        ],
    )(_sc_body)

    g = sck(scores, e_flat)

    lg = pl.pallas_call(
        _contract_body,
        grid=(B,),
        in_specs=[
            pl.BlockSpec((K, D), lambda b: (b, 0)),
            pl.BlockSpec((K, G, D), lambda b: (0, 0, 0)),
            pl.BlockSpec((K, G), lambda b: (0, 0)),
        ],
        out_specs=pl.BlockSpec((1, K, G), lambda b: (b, 0, 0)),
        out_shape=jax.ShapeDtypeStruct((B, K, G), jnp.float32),
    )(g, c_weight, cls_bias)

    return lg.reshape(B, K * G)[:, :NUM_CLASSES]


# speedup vs baseline: 1.3652x; 1.0152x over previous
import functools

import jax
import jax.numpy as jnp
from jax import lax
from jax.experimental import pallas as pl
from jax.experimental.pallas import tpu as pltpu
from jax.experimental.pallas import tpu_sc as plsc

B, T, D = 2, 2048, 768
K = 16
NUM_CLASSES = 1000
G = 63
TOP_M = 8

NCHUNK = T // 16
DCHUNK = D // 16


def _scores_body(e_ref, q_ref, out_ref):
    E2 = e_ref[0]
    sq = jnp.where(q_ref[...] >= 0, 1.0, -1.0).astype(jnp.float32)
    out_ref[...] = lax.dot_general(
        sq, E2, (((1,), (1,)), ((), ())),
        preferred_element_type=jnp.float32,
        precision=lax.Precision.HIGHEST,
    )


def _sc_body(scores_hbm, e_hbm, g_hbm, s_v, idx_v, rows_v, g_v, sem_g):
    cid = lax.axis_index("c")
    sid = lax.axis_index("s")
    lanes = lax.broadcasted_iota(jnp.int32, (16,), 0)
    NEG = jnp.float32(-jnp.inf)
    BIG = jnp.int32(1 << 30)
    _sc_one(scores_hbm, e_hbm, g_hbm, s_v, idx_v, rows_v, g_v, sem_g,
            cid, sid, lanes, NEG, BIG)


def _sc_one(scores_hbm, e_hbm, g_hbm, s_v, idx_v, rows_v, g_v, sem_g,
            bb, sid, lanes, NEG, BIG):
    w = bb * K + sid

    pltpu.sync_copy(scores_hbm.at[w], s_v)

    NACC = 8

    def combine(xv, xi, yv, yi):
        idx = jnp.where(
            yv > xv, yi, jnp.where(xv > yv, xi, jnp.minimum(xi, yi))
        )
        return jnp.maximum(xv, yv), idx

    idx_acc = jnp.full((16,), bb * T, jnp.int32)
    for m in range(TOP_M):
        def fold(jj, carry):
            out = list(carry)
            for a in range(NACC):
                mval, midx = out[2 * a], out[2 * a + 1]
                off = jj * (16 * NACC) + a * 16
                v = s_v[pl.ds(off, 16)]
                upd = v > mval
                out[2 * a] = jnp.where(upd, v, mval)
                out[2 * a + 1] = jnp.where(upd, off + lanes, midx)
            return tuple(out)

        init = []
        for a in range(NACC):
            init += [jnp.full((16,), NEG, jnp.float32),
                     jnp.full((16,), BIG, jnp.int32)]
        res = lax.fori_loop(0, NCHUNK // NACC, fold, tuple(init))
        vs = [res[2 * a] for a in range(NACC)]
        is_ = [res[2 * a + 1] for a in range(NACC)]
        while len(vs) > 1:
            nv, ni = [], []
            for a in range(0, len(vs), 2):
                cv, ci = combine(vs[a], is_[a], vs[a + 1], is_[a + 1])
                nv.append(cv)
                ni.append(ci)
            vs, is_ = nv, ni
        mval, midx = vs[0], is_[0]
        mval, midx = combine(
            mval, midx, lax.rev(mval, (0,)), lax.rev(midx, (0,))
        )
        mx = mval[0]
        tidx = midx[0]
        for l in range(1, 8):
            v = mval[l]
            i = midx[l]
            take = (v > mx) | ((v == mx) & (i < tidx))
            mx = jnp.where(take, v, mx)
            tidx = jnp.where(take, i, tidx)
        jc = tidx // 16
        lc = tidx - jc * 16
        chunk = s_v[pl.ds(jc * 16, 16)]
        s_v[pl.ds(jc * 16, 16)] = jnp.where(lanes == lc, NEG, chunk)
        idx_acc = jnp.where(lanes == m, bb * T + tidx, idx_acc)

    idx_v[...] = idx_acc
    pltpu.async_copy(e_hbm.at[idx_v.at[pl.ds(0, TOP_M)]], rows_v, sem_g).wait()

    for j in range(DCHUNK):
        acc = rows_v[0, pl.ds(j * 16, 16)]
        for r in range(1, TOP_M):
            acc = acc + rows_v[r, pl.ds(j * 16, 16)]
        g_v[pl.ds(j * 16, 16)] = acc

    pltpu.sync_copy(g_v, g_hbm.at[w])


def _contract_body(g_ref, c_ref, cb_ref, out_ref):
    g = g_ref[...]
    signed = jnp.where(c_ref[...] >= 0, g[:, None, :], -g[:, None, :])
    out_ref[0] = jnp.sum(signed, axis=-1) + cb_ref[...]


def kernel(E, q_weight, c_weight, q_bias, cls_bias):
    del q_bias
    scores = pl.pallas_call(
        _scores_body,
        grid=(B,),
        in_specs=[
            pl.BlockSpec((1, T, D), lambda b: (b, 0, 0)),
            pl.BlockSpec((K, D), lambda b: (0, 0)),
        ],
        out_specs=pl.BlockSpec((K, T), lambda b: (b, 0)),
        out_shape=jax.ShapeDtypeStruct((B * K, T), jnp.float32),
    )(E, q_weight)

    e_flat = E.reshape(B * T, D)

    mesh = plsc.VectorSubcoreMesh(core_axis_name="c", subcore_axis_name="s")
    sck = functools.partial(
        pl.kernel,
        mesh=mesh,
        out_type=jax.ShapeDtypeStruct((B * K, D), jnp.float32),
        scratch_types=[
            pltpu.VMEM((T,), jnp.float32),
            pltpu.VMEM((16,), jnp.int32),
            pltpu.VMEM((TOP_M, D), jnp.float32),
            pltpu.VMEM((D,), jnp.float32),
            pltpu.SemaphoreType.DMA,
        ],
    )(_sc_body)

    g = sck(scores, e_flat)

    lg = pl.pallas_call(
        _contract_body,
        grid=(B,),
        in_specs=[
            pl.BlockSpec((K, D), lambda b: (b, 0)),
            pl.BlockSpec((K, G, D), lambda b: (0, 0, 0)),
            pl.BlockSpec((K, G), lambda b: (0, 0)),
        ],
        out_specs=pl.BlockSpec((1, K, G), lambda b: (b, 0, 0)),
        out_shape=jax.ShapeDtypeStruct((B, K, G), jnp.float32),
    )(g, c_weight, cls_bias)

    return lg.reshape(B, K * G)[:, :NUM_CLASSES]


# speedup vs baseline: 1.5521x; 1.1369x over previous
import functools

import jax
import jax.numpy as jnp
from jax import lax
from jax.experimental import pallas as pl
from jax.experimental.pallas import tpu as pltpu
from jax.experimental.pallas import tpu_sc as plsc

B, T, D = 2, 2048, 768
K = 16
NUM_CLASSES = 1000
G = 63
TOP_M = 8

NCHUNK = T // 16
DCHUNK = D // 16


def _scores_body(e_ref, q_ref, out_ref):
    E2 = e_ref[0]
    sq = jnp.where(q_ref[...] >= 0, 1.0, -1.0).astype(jnp.bfloat16)
    e_hi = E2.astype(jnp.bfloat16)
    r1 = E2 - e_hi.astype(jnp.float32)
    e_mid = r1.astype(jnp.bfloat16)
    e_lo = (r1 - e_mid.astype(jnp.float32)).astype(jnp.bfloat16)

    def dot(e_piece):
        return lax.dot_general(
            sq, e_piece, (((1,), (1,)), ((), ())),
            preferred_element_type=jnp.float32,
        )

    out_ref[...] = dot(e_hi) + dot(e_mid) + dot(e_lo)


def _sc_body(scores_hbm, e_hbm, g_hbm, s_v, idx_v, rows_v, g_v, sem_g):
    cid = lax.axis_index("c")
    sid = lax.axis_index("s")
    lanes = lax.broadcasted_iota(jnp.int32, (16,), 0)
    NEG = jnp.float32(-jnp.inf)
    BIG = jnp.int32(1 << 30)
    _sc_one(scores_hbm, e_hbm, g_hbm, s_v, idx_v, rows_v, g_v, sem_g,
            cid, sid, lanes, NEG, BIG)


def _sc_one(scores_hbm, e_hbm, g_hbm, s_v, idx_v, rows_v, g_v, sem_g,
            bb, sid, lanes, NEG, BIG):
    w = bb * K + sid

    pltpu.sync_copy(scores_hbm.at[w], s_v)

    NACC = 8

    def combine(xv, xi, yv, yi):
        idx = jnp.where(
            yv > xv, yi, jnp.where(xv > yv, xi, jnp.minimum(xi, yi))
        )
        return jnp.maximum(xv, yv), idx

    idx_acc = jnp.full((16,), bb * T, jnp.int32)
    for m in range(TOP_M):
        def fold(jj, carry):
            out = list(carry)
            for a in range(NACC):
                mval, midx = out[2 * a], out[2 * a + 1]
                off = jj * (16 * NACC) + a * 16
                v = s_v[pl.ds(off, 16)]
                upd = v > mval
                out[2 * a] = jnp.where(upd, v, mval)
                out[2 * a + 1] = jnp.where(upd, off + lanes, midx)
            return tuple(out)

        init = []
        for a in range(NACC):
            init += [jnp.full((16,), NEG, jnp.float32),
                     jnp.full((16,), BIG, jnp.int32)]
        res = lax.fori_loop(0, NCHUNK // NACC, fold, tuple(init))
        vs = [res[2 * a] for a in range(NACC)]
        is_ = [res[2 * a + 1] for a in range(NACC)]
        while len(vs) > 1:
            nv, ni = [], []
            for a in range(0, len(vs), 2):
                cv, ci = combine(vs[a], is_[a], vs[a + 1], is_[a + 1])
                nv.append(cv)
                ni.append(ci)
            vs, is_ = nv, ni
        mval, midx = vs[0], is_[0]
        mval, midx = combine(
            mval, midx, lax.rev(mval, (0,)), lax.rev(midx, (0,))
        )
        mx = mval[0]
        tidx = midx[0]
        for l in range(1, 8):
            v = mval[l]
            i = midx[l]
            take = (v > mx) | ((v == mx) & (i < tidx))
            mx = jnp.where(take, v, mx)
            tidx = jnp.where(take, i, tidx)
        jc = tidx // 16
        lc = tidx - jc * 16
        chunk = s_v[pl.ds(jc * 16, 16)]
        s_v[pl.ds(jc * 16, 16)] = jnp.where(lanes == lc, NEG, chunk)
        idx_acc = jnp.where(lanes == m, bb * T + tidx, idx_acc)

    idx_v[...] = idx_acc
    pltpu.async_copy(e_hbm.at[idx_v.at[pl.ds(0, TOP_M)]], rows_v, sem_g).wait()

    for j in range(DCHUNK):
        acc = rows_v[0, pl.ds(j * 16, 16)]
        for r in range(1, TOP_M):
            acc = acc + rows_v[r, pl.ds(j * 16, 16)]
        g_v[pl.ds(j * 16, 16)] = acc

    pltpu.sync_copy(g_v, g_hbm.at[w])


def _contract_body(g_ref, c_ref, cb_ref, out_ref):
    g = g_ref[...]
    signed = jnp.where(c_ref[...] >= 0, g[:, None, :], -g[:, None, :])
    out_ref[0] = jnp.sum(signed, axis=-1) + cb_ref[...]


def kernel(E, q_weight, c_weight, q_bias, cls_bias):
    del q_bias
    scores = pl.pallas_call(
        _scores_body,
        grid=(B,),
        in_specs=[
            pl.BlockSpec((1, T, D), lambda b: (b, 0, 0)),
            pl.BlockSpec((K, D), lambda b: (0, 0)),
        ],
        out_specs=pl.BlockSpec((K, T), lambda b: (b, 0)),
        out_shape=jax.ShapeDtypeStruct((B * K, T), jnp.float32),
    )(E, q_weight)

    e_flat = E.reshape(B * T, D)

    mesh = plsc.VectorSubcoreMesh(core_axis_name="c", subcore_axis_name="s")
    sck = functools.partial(
        pl.kernel,
        mesh=mesh,
        out_type=jax.ShapeDtypeStruct((B * K, D), jnp.float32),
        scratch_types=[
            pltpu.VMEM((T,), jnp.float32),
            pltpu.VMEM((16,), jnp.int32),
            pltpu.VMEM((TOP_M, D), jnp.float32),
            pltpu.VMEM((D,), jnp.float32),
            pltpu.SemaphoreType.DMA,
        ],
    )(_sc_body)

    g = sck(scores, e_flat)

    lg = pl.pallas_call(
        _contract_body,
        grid=(B,),
        in_specs=[
            pl.BlockSpec((K, D), lambda b: (b, 0)),
            pl.BlockSpec((K, G, D), lambda b: (0, 0, 0)),
            pl.BlockSpec((K, G), lambda b: (0, 0)),
        ],
        out_specs=pl.BlockSpec((1, K, G), lambda b: (b, 0, 0)),
        out_shape=jax.ShapeDtypeStruct((B, K, G), jnp.float32),
    )(g, c_weight, cls_bias)

    return lg.reshape(B, K * G)[:, :NUM_CLASSES]
